# Initial kernel scaffold; baseline (speedup 1.0000x reference)
import jax
import jax.numpy as jnp
from jax.experimental import pallas as pl


def kernel(idx, table):
    raise NotImplementedError("write your pallas kernel here")


# speedup vs baseline: 1.0221x; 1.0221x over previous
import functools

import jax
import jax.numpy as jnp
from jax import lax
from jax.experimental import pallas as pl
from jax.experimental.pallas import tpu as pltpu
from jax.experimental.pallas import tpu_sc as plsc

L = 128
D = 32


def _make_gather(n_rows: int, n_chunks: int):
    info = plsc.get_sparse_core_info()
    nw = info.num_cores * info.num_subcores
    rows_per_w = n_chunks // nw
    mesh = plsc.VectorSubcoreMesh(core_axis_name="c", subcore_axis_name="s")

    @functools.partial(
        pl.kernel,
        out_type=jax.ShapeDtypeStruct((n_rows, D), jnp.float32),
        mesh=mesh,
        scratch_types=[
            pltpu.VMEM((rows_per_w, L), jnp.int32),
            pltpu.VMEM((L, D), jnp.float32),
            pltpu.SemaphoreType.DMA,
        ],
        compiler_params=pltpu.CompilerParams(use_tc_tiling_on_sc=False),
    )
    def k(idx_hbm, table_hbm, out_hbm, idx_v, rows_v, sem):
        wid = lax.axis_index("s") * info.num_cores + lax.axis_index("c")
        base = wid * rows_per_w
        pltpu.sync_copy(idx_hbm.at[pl.ds(base, rows_per_w)], idx_v)

        @pl.loop(0, rows_per_w)
        def _(j):
            pltpu.async_copy(table_hbm.at[idx_v.at[j]], rows_v, sem).wait()
            pltpu.sync_copy(rows_v, out_hbm.at[pl.ds((base + j) * L, L)])

    return k


def kernel(idx, table):
    b0, b1 = idx.shape
    n = b0 * b1
    idx2 = idx.reshape(n // L, L).astype(jnp.int32)
    out = _make_gather(n, n // L)(idx2, table)
    return out.reshape(b0, b1, D)


# speedup vs baseline: 1.1106x; 1.0866x over previous
import functools

import jax
import jax.numpy as jnp
from jax import lax
from jax.experimental import pallas as pl
from jax.experimental.pallas import tpu as pltpu
from jax.experimental.pallas import tpu_sc as plsc

L = 128
D = 32
NBUF = 8
M = 4


def _make_gather(n_rows: int, n_chunks: int):
    info = plsc.get_sparse_core_info()
    nw = info.num_cores * info.num_subcores
    rows_per_w = n_chunks // nw
    assert rows_per_w % NBUF == 0 and rows_per_w >= 3 * NBUF
    mesh = plsc.VectorSubcoreMesh(core_axis_name="c", subcore_axis_name="s")

    @functools.partial(
        pl.kernel,
        out_type=jax.ShapeDtypeStruct((n_rows, D), jnp.float32),
        mesh=mesh,
        scratch_types=[
            pltpu.VMEM((rows_per_w, L), jnp.int32),
            pltpu.VMEM((NBUF, L, D), jnp.float32),
        ]
        + [pltpu.SemaphoreType.DMA] * (2 * NBUF),
        compiler_params=pltpu.CompilerParams(use_tc_tiling_on_sc=False),
    )
    def k(idx_hbm, table_hbm, out_hbm, idx_v, rows_v, *sems):
        gsem = sems[:NBUF]
        ssem = sems[NBUF:]
        wid = lax.axis_index("s") * info.num_cores + lax.axis_index("c")
        base = wid * rows_per_w
        pltpu.sync_copy(idx_hbm.at[pl.ds(base, rows_per_w)], idx_v)

        def g_start(jj, b):
            pltpu.async_copy(table_hbm.at[idx_v.at[jj]], rows_v.at[b], gsem[b])

        def g_wait(jj, b):
            pltpu.make_async_copy(
                table_hbm.at[idx_v.at[jj]], rows_v.at[b], gsem[b]
            ).wait()

        def s_start(jj, b):
            pltpu.async_copy(
                rows_v.at[b], out_hbm.at[pl.ds((base + jj) * L, L)], ssem[b]
            )

        def s_wait(jj, b):
            pltpu.make_async_copy(
                rows_v.at[b], out_hbm.at[pl.ds((base + jj) * L, L)], ssem[b]
            ).wait()

        def step(jj, b, issue_next, wait_prev_store):
            g_wait(jj, b)
            s_start(jj, b)
            if issue_next:
                b2 = (b + M) % NBUF
                if wait_prev_store:
                    s_wait(jj + M - NBUF, b2)
                g_start(jj + M, b2)

        for kk in range(M):
            g_start(kk, kk)

        for b in range(NBUF):
            step(b, b, issue_next=True, wait_prev_store=(b + M >= NBUF))

        @pl.loop(NBUF, rows_per_w - NBUF, step=NBUF)
        def _(j):
            for b in range(NBUF):
                step(j + b, b, issue_next=True, wait_prev_store=True)

        for b in range(NBUF):
            jj = rows_per_w - NBUF + b
            step(jj, b, issue_next=(b < NBUF - M), wait_prev_store=True)

        for b in range(NBUF):
            s_wait(rows_per_w - NBUF + b, b)

    return k


def kernel(idx, table):
    b0, b1 = idx.shape
    n = b0 * b1
    idx2 = idx.reshape(n // L, L).astype(jnp.int32)
    out = _make_gather(n, n // L)(idx2, table)
    return out.reshape(b0, b1, D)
